# Initial kernel scaffold; baseline (speedup 1.0000x reference)
import jax
import jax.numpy as jnp
from jax.experimental import pallas as pl


def kernel(indices, sim_mat):
    raise NotImplementedError("write your pallas kernel here")


# speedup vs baseline: 3.2348x; 3.2348x over previous
import dataclasses
import functools

import jax
import jax.numpy as jnp
from jax import lax
from jax.experimental import pallas as pl
from jax.experimental.pallas import tpu as pltpu
from jax.experimental.pallas import tpu_sc as plsc

_DIM = 8192
_B = 4096
_NC = 2
_NS = 16
_NW = _NC * _NS
_RPW = _B // _NW
_KB = 8
_NBATCH = _RPW // _KB
_L = 16


def _sim_body(idx_hbm, sim_hbm, out_hbm, idx_v, rows_v, outb_v, sem):
    cid = lax.axis_index("c")
    sid = lax.axis_index("s")
    wid = sid * _NC + cid
    base = wid * _RPW
    pltpu.sync_copy(idx_hbm, idx_v)

    @pl.loop(0, _NBATCH)
    def _batch(b):
        row0 = base + b * _KB
        row_ids = idx_v.at[pl.ds(row0, _KB)]
        pltpu.async_copy(sim_hbm.at[row_ids], rows_v, sem).wait()

        @pl.loop(0, _B // _L)
        def _chunk(c):
            cols = idx_v[pl.ds(c * _L, _L)]
            for r in range(_KB):
                rsplat = jnp.full((_L,), r, jnp.int32)
                vals = plsc.load_gather(rows_v, [rsplat, cols])
                outb_v[r, pl.ds(c * _L, _L)] = vals

        pltpu.sync_copy(outb_v, out_hbm.at[pl.ds(row0, _KB)])


def kernel(indices, sim_mat):
    indices = indices.astype(jnp.int32)
    cp = pltpu.CompilerParams()
    if "needs_layout_passes" in pltpu.CompilerParams.__dataclass_fields__:
        cp = dataclasses.replace(cp, needs_layout_passes=False)
    mesh = plsc.VectorSubcoreMesh(core_axis_name="c", subcore_axis_name="s")
    k = pl.kernel(
        _sim_body,
        out_type=jax.ShapeDtypeStruct((_B, _B), jnp.float32),
        mesh=mesh,
        compiler_params=cp,
        scratch_types=[
            pltpu.VMEM((_B,), jnp.int32),
            pltpu.VMEM((_KB, _DIM), jnp.float32),
            pltpu.VMEM((_KB, _B), jnp.float32),
            pltpu.SemaphoreType.DMA,
        ],
    )
    return k(indices, sim_mat)


# speedup vs baseline: 3.6568x; 1.1305x over previous
import dataclasses
import functools

import jax
import jax.numpy as jnp
from jax import lax
from jax.experimental import pallas as pl
from jax.experimental.pallas import tpu as pltpu
from jax.experimental.pallas import tpu_sc as plsc

_DIM = 8192
_B = 4096
_NC = 2
_NS = 16
_NW = _NC * _NS
_RPW = _B // _NW
_KB = 4
_NBATCH = _RPW // _KB
_L = 16


def _sim_body(idx_hbm, idx2_hbm, sim_hbm, out_hbm,
              idx_v, rid_v, rows0, rows1, out0, out1,
              gsem0, gsem1, wsem0, wsem1):
    cid = lax.axis_index("c")
    sid = lax.axis_index("s")
    wid = sid * _NC + cid
    base = wid * _RPW
    pltpu.sync_copy(idx_hbm, idx_v)
    pltpu.sync_copy(idx2_hbm.at[pl.ds(wid * _NBATCH, _NBATCH)], rid_v)

    rows = (rows0, rows1)
    outs = (out0, out1)
    gsems = (gsem0, gsem1)
    wsems = (wsem0, wsem1)

    def start_gather(b, p):
        pltpu.async_copy(sim_hbm.at[rid_v.at[b]], rows[p], gsems[p])

    def wait_gather(p):
        pltpu.make_async_copy(sim_hbm.at[pl.ds(0, _KB)], rows[p],
                              gsems[p]).wait()

    def start_write(b, p):
        pltpu.async_copy(outs[p], out_hbm.at[pl.ds(base + b * _KB, _KB)],
                         wsems[p])

    def wait_write(p):
        pltpu.make_async_copy(outs[p], out_hbm.at[pl.ds(0, _KB)],
                              wsems[p]).wait()

    start_gather(0, 0)
    start_gather(1, 1)

    @pl.loop(0, _NBATCH, step=2)
    def _pair(b0):
        for p in range(2):
            b = b0 + p
            wait_gather(p)

            @pl.when(b >= 2)
            def _():
                wait_write(p)

            @pl.loop(0, _B // _L)
            def _chunk(c):
                cols = idx_v[pl.ds(c * _L, _L)]
                for r in range(_KB):
                    rsplat = jnp.full((_L,), r, jnp.int32)
                    vals = plsc.load_gather(rows[p], [rsplat, cols])
                    outs[p][r, pl.ds(c * _L, _L)] = vals

            start_write(b, p)

            @pl.when(b + 2 < _NBATCH)
            def _():
                start_gather(b + 2, p)

    wait_write(0)
    wait_write(1)


def kernel(indices, sim_mat):
    indices = indices.astype(jnp.int32)
    idx2 = indices.reshape(_B // _KB, _KB)
    cp = pltpu.CompilerParams()
    if "needs_layout_passes" in pltpu.CompilerParams.__dataclass_fields__:
        cp = dataclasses.replace(cp, needs_layout_passes=False)
    mesh = plsc.VectorSubcoreMesh(core_axis_name="c", subcore_axis_name="s")
    k = pl.kernel(
        _sim_body,
        out_type=jax.ShapeDtypeStruct((_B, _B), jnp.float32),
        mesh=mesh,
        compiler_params=cp,
        scratch_types=[
            pltpu.VMEM((_B,), jnp.int32),
            pltpu.VMEM((_NBATCH, _KB), jnp.int32),
            pltpu.VMEM((_KB, _DIM), jnp.float32),
            pltpu.VMEM((_KB, _DIM), jnp.float32),
            pltpu.VMEM((_KB, _B), jnp.float32),
            pltpu.VMEM((_KB, _B), jnp.float32),
            pltpu.SemaphoreType.DMA,
            pltpu.SemaphoreType.DMA,
            pltpu.SemaphoreType.DMA,
            pltpu.SemaphoreType.DMA,
        ],
    )
    return k(indices, idx2, sim_mat)


# speedup vs baseline: 9.0094x; 2.4637x over previous
import dataclasses
import functools

import jax
import jax.numpy as jnp
from jax import lax
from jax.experimental import pallas as pl
from jax.experimental.pallas import tpu as pltpu
from jax.experimental.pallas import tpu_sc as plsc

_DIM = 8192
_B = 4096
_NC = 2
_NS = 16
_NW = _NC * _NS
_RPW = _B // _NW
_KB = 4
_NBATCH = _RPW // _KB
_L = 16


def _sim_body(idx_hbm, idx2_hbm, sim_hbm, out_hbm,
              idx_v, rid_v, rows0, rows1, out0, out1,
              gsem0, gsem1, wsem0, wsem1):
    cid = lax.axis_index("c")
    sid = lax.axis_index("s")
    wid = sid * _NC + cid
    base = wid * _RPW
    pltpu.sync_copy(idx_hbm, idx_v)
    pltpu.sync_copy(idx2_hbm.at[pl.ds(wid * _NBATCH, _NBATCH)], rid_v)

    rows = (rows0, rows1)
    outs = (out0, out1)
    gsems = (gsem0, gsem1)
    wsems = (wsem0, wsem1)

    def start_gather(b, p):
        pltpu.async_copy(sim_hbm.at[rid_v.at[b]], rows[p], gsems[p])

    def wait_gather(p):
        pltpu.make_async_copy(sim_hbm.at[pl.ds(0, _KB)], rows[p],
                              gsems[p]).wait()

    def start_write(b, p):
        pltpu.async_copy(outs[p], out_hbm.at[pl.ds(base + b * _KB, _KB)],
                         wsems[p])

    def wait_write(p):
        pltpu.make_async_copy(outs[p], out_hbm.at[pl.ds(0, _KB)],
                              wsems[p]).wait()

    start_gather(0, 0)
    start_gather(1, 1)

    @pl.loop(0, _NBATCH, step=2)
    def _pair(b0):
        for p in range(2):
            b = b0 + p
            wait_gather(p)

            @pl.when(b >= 2)
            def _():
                wait_write(p)

            @plsc.parallel_loop(0, _B // _L, unroll=4)
            def _chunk(c):
                cols = idx_v[pl.ds(c * _L, _L)]
                for r in range(_KB):
                    rsplat = jnp.full((_L,), r, jnp.int32)
                    vals = plsc.load_gather(rows[p], [rsplat, cols])
                    outs[p][r, pl.ds(c * _L, _L)] = vals

            start_write(b, p)

            @pl.when(b + 2 < _NBATCH)
            def _():
                start_gather(b + 2, p)

    wait_write(0)
    wait_write(1)


def kernel(indices, sim_mat):
    indices = indices.astype(jnp.int32)
    idx2 = indices.reshape(_B // _KB, _KB)
    cp = pltpu.CompilerParams()
    if "needs_layout_passes" in pltpu.CompilerParams.__dataclass_fields__:
        cp = dataclasses.replace(cp, needs_layout_passes=False)
    mesh = plsc.VectorSubcoreMesh(core_axis_name="c", subcore_axis_name="s")
    k = pl.kernel(
        _sim_body,
        out_type=jax.ShapeDtypeStruct((_B, _B), jnp.float32),
        mesh=mesh,
        compiler_params=cp,
        scratch_types=[
            pltpu.VMEM((_B,), jnp.int32),
            pltpu.VMEM((_NBATCH, _KB), jnp.int32),
            pltpu.VMEM((_KB, _DIM), jnp.float32),
            pltpu.VMEM((_KB, _DIM), jnp.float32),
            pltpu.VMEM((_KB, _B), jnp.float32),
            pltpu.VMEM((_KB, _B), jnp.float32),
            pltpu.SemaphoreType.DMA,
            pltpu.SemaphoreType.DMA,
            pltpu.SemaphoreType.DMA,
            pltpu.SemaphoreType.DMA,
        ],
    )
    return k(indices, idx2, sim_mat)


# speedup vs baseline: 9.0206x; 1.0012x over previous
import dataclasses
import functools

import jax
import jax.numpy as jnp
from jax import lax
from jax.experimental import pallas as pl
from jax.experimental.pallas import tpu as pltpu
from jax.experimental.pallas import tpu_sc as plsc

_DIM = 8192
_B = 4096
_NC = 2
_NS = 16
_NW = _NC * _NS
_RPW = _B // _NW
_KB = 4
_NBATCH = _RPW // _KB
_L = 16


def _sim_body(idx_hbm, idx2_hbm, sim_hbm, out_hbm,
              idx_v, rid_v, rows0, rows1, out0, out1,
              gsem0, gsem1, wsem0, wsem1):
    cid = lax.axis_index("c")
    sid = lax.axis_index("s")
    wid = sid * _NC + cid
    base = wid * _RPW
    pltpu.sync_copy(idx_hbm, idx_v)
    pltpu.sync_copy(idx2_hbm.at[pl.ds(wid * _NBATCH, _NBATCH)], rid_v)

    rows = (rows0, rows1)
    outs = (out0, out1)
    gsems = (gsem0, gsem1)
    wsems = (wsem0, wsem1)

    def start_gather(b, p):
        pltpu.async_copy(sim_hbm.at[rid_v.at[b]], rows[p], gsems[p])

    def wait_gather(p):
        pltpu.make_async_copy(sim_hbm.at[pl.ds(0, _KB)], rows[p],
                              gsems[p]).wait()

    def start_write(b, p):
        pltpu.async_copy(outs[p], out_hbm.at[pl.ds(base + b * _KB, _KB)],
                         wsems[p])

    def wait_write(p):
        pltpu.make_async_copy(outs[p], out_hbm.at[pl.ds(0, _KB)],
                              wsems[p]).wait()

    start_gather(0, 0)
    start_gather(1, 1)

    @pl.loop(0, _NBATCH, step=2)
    def _pair(b0):
        for p in range(2):
            b = b0 + p
            wait_gather(p)

            @pl.when(b >= 2)
            def _():
                wait_write(p)

            @plsc.parallel_loop(0, _B // _L, unroll=8)
            def _chunk(c):
                cols = idx_v[pl.ds(c * _L, _L)]
                for r in range(_KB):
                    rsplat = jnp.full((_L,), r, jnp.int32)
                    vals = plsc.load_gather(rows[p], [rsplat, cols])
                    outs[p][r, pl.ds(c * _L, _L)] = vals

            start_write(b, p)

            @pl.when(b + 2 < _NBATCH)
            def _():
                start_gather(b + 2, p)

    wait_write(0)
    wait_write(1)


def kernel(indices, sim_mat):
    indices = indices.astype(jnp.int32)
    idx2 = indices.reshape(_B // _KB, _KB)
    cp = pltpu.CompilerParams()
    if "needs_layout_passes" in pltpu.CompilerParams.__dataclass_fields__:
        cp = dataclasses.replace(cp, needs_layout_passes=False)
    mesh = plsc.VectorSubcoreMesh(core_axis_name="c", subcore_axis_name="s")
    k = pl.kernel(
        _sim_body,
        out_type=jax.ShapeDtypeStruct((_B, _B), jnp.float32),
        mesh=mesh,
        compiler_params=cp,
        scratch_types=[
            pltpu.VMEM((_B,), jnp.int32),
            pltpu.VMEM((_NBATCH, _KB), jnp.int32),
            pltpu.VMEM((_KB, _DIM), jnp.float32),
            pltpu.VMEM((_KB, _DIM), jnp.float32),
            pltpu.VMEM((_KB, _B), jnp.float32),
            pltpu.VMEM((_KB, _B), jnp.float32),
            pltpu.SemaphoreType.DMA,
            pltpu.SemaphoreType.DMA,
            pltpu.SemaphoreType.DMA,
            pltpu.SemaphoreType.DMA,
        ],
    )
    return k(indices, idx2, sim_mat)
